# Initial kernel scaffold; baseline (speedup 1.0000x reference)
import jax
import jax.numpy as jnp
from jax.experimental import pallas as pl


def kernel(inputs, embed_table, W1, b1, W2, b2):
    raise NotImplementedError("write your pallas kernel here")


# speedup vs baseline: 8.0474x; 8.0474x over previous
import functools

import jax
import jax.numpy as jnp
from jax import lax
from jax.experimental import pallas as pl
from jax.experimental.pallas import tpu as pltpu
from jax.experimental.pallas import tpu_sc as plsc

DIM_EMB = 32
DIM_HID = 64
DIM_OUT = 2

NUM_SC = 2
NUM_SUBCORES = 16
NW = NUM_SC * NUM_SUBCORES
CHUNK = 128


def _make_gather(n_rows: int, d: int):
    assert n_rows % (NW * CHUNK) == 0
    rows_per_tile = n_rows // NW
    chunks_per_tile = rows_per_tile // CHUNK

    mesh = plsc.VectorSubcoreMesh(core_axis_name="c", subcore_axis_name="s")

    @functools.partial(
        pl.kernel,
        mesh=mesh,
        out_type=jax.ShapeDtypeStruct((n_rows, d), jnp.float32),
        scratch_types=[
            pltpu.VMEM((chunks_per_tile, CHUNK), jnp.int32),
            pltpu.VMEM((CHUNK, d), jnp.float32),
            pltpu.SemaphoreType.DMA,
        ],
        compiler_params=pltpu.CompilerParams(use_tc_tiling_on_sc=False),
    )
    def gather_kernel(idx_hbm, table_hbm, out_hbm, idx_v, rows_v, sem):
        wid = lax.axis_index("s") * NUM_SC + lax.axis_index("c")
        chunk_base = wid * chunks_per_tile
        row_base = wid * rows_per_tile
        pltpu.sync_copy(idx_hbm.at[pl.ds(chunk_base, chunks_per_tile)], idx_v)

        def body(j, carry):
            pltpu.async_copy(table_hbm.at[idx_v.at[j]], rows_v, sem).wait()
            pltpu.sync_copy(rows_v, out_hbm.at[pl.ds(row_base + j * CHUNK, CHUNK)])
            return carry

        lax.fori_loop(0, chunks_per_tile, body, 0)

    return gather_kernel


def _mlp_body(x_ref, w1_ref, b1_ref, w2_ref, b2_ref, o_ref):
    x = x_ref[...]
    h = jnp.dot(x, w1_ref[...], preferred_element_type=jnp.float32) + b1_ref[...]
    h = jnp.maximum(h, 0.0)
    o = jnp.dot(h, w2_ref[...], preferred_element_type=jnp.float32) + b2_ref[...]
    m = jnp.max(o, axis=1, keepdims=True)
    s = o - m
    lse = jnp.log(jnp.sum(jnp.exp(s), axis=1, keepdims=True))
    o_ref[...] = s - lse


def kernel(inputs, embed_table, W1, b1, W2, b2):
    batch, ctx = inputs.shape
    n_rows = batch * ctx
    feat = ctx * DIM_EMB

    idx2d = inputs.reshape(-1, CHUNK).astype(jnp.int32)
    embds_flat = _make_gather(n_rows, DIM_EMB)(idx2d, embed_table)
    embds = embds_flat.reshape(batch, feat)

    tb = 512
    out = pl.pallas_call(
        _mlp_body,
        grid=(batch // tb,),
        in_specs=[
            pl.BlockSpec((tb, feat), lambda i: (i, 0)),
            pl.BlockSpec((feat, DIM_HID), lambda i: (0, 0)),
            pl.BlockSpec((1, DIM_HID), lambda i: (0, 0)),
            pl.BlockSpec((DIM_HID, DIM_OUT), lambda i: (0, 0)),
            pl.BlockSpec((1, DIM_OUT), lambda i: (0, 0)),
        ],
        out_specs=pl.BlockSpec((tb, DIM_OUT), lambda i: (i, 0)),
        out_shape=jax.ShapeDtypeStruct((batch, DIM_OUT), jnp.float32),
    )(embds, W1, b1.reshape(1, DIM_HID), W2, b2.reshape(1, DIM_OUT))
    return out
